# Initial kernel scaffold; baseline (speedup 1.0000x reference)
import jax
import jax.numpy as jnp
from jax.experimental import pallas as pl


def kernel(x, edge_attr_ipv6, edge_attr_tsch, edge_index_ipv6, edge_index_tsch, batch, W_ipv6, a_src_ipv6, a_dst_ipv6, b_ipv6, W_tsch, a_src_tsch, a_dst_tsch, b_tsch, Wf_ipv6, bf_ipv6, Wf_tsch, bf_tsch, W1, b1, W2, b2, W3, b3, We, be, Wv, bv):
    raise NotImplementedError("write your pallas kernel here")


# speedup vs baseline: 16.5550x; 16.5550x over previous
import functools

import jax
import jax.numpy as jnp
from jax import lax
from jax.experimental import pallas as pl
from jax.experimental.pallas import tpu as pltpu
from jax.experimental.pallas import tpu_sc as plsc

N = 10000
E = 320000
D = 128
H = 8
C = 64
G = 64
HC = H * C

EE = E + N
NC = 2
NS = 16
NW = NC * NS
CHUNK = 128
CPT = -(-EE // (NW * CHUNK))
EEP = NW * CHUNK * CPT
R2 = 10112
ROWS_PT = R2 // NS

_f32 = jnp.float32


def _mesh():
  return plsc.VectorSubcoreMesh(
      core_axis_name="c", subcore_axis_name="s",
      num_cores=NC, num_subcores=NS)


def _wid():
  return lax.axis_index("s") * NC + lax.axis_index("c")


def _make_gather(kc):
  @functools.partial(
      pl.kernel,
      out_type=jax.ShapeDtypeStruct((EEP, kc), _f32),
      mesh=_mesh(),
      scratch_types=[
          pltpu.VMEM((CHUNK,), jnp.int32),
          pltpu.VMEM((CHUNK, kc), _f32),
          pltpu.SemaphoreType.DMA,
      ],
  )
  def gk(table, idx, out, idx_v, rows_v, sem):
    base = _wid() * (CPT * CHUNK)

    def body(j, carry):
      off = base + j * CHUNK
      pltpu.sync_copy(idx.at[pl.ds(off, CHUNK)], idx_v)
      pltpu.async_copy(table.at[idx_v], rows_v, sem).wait()
      pltpu.sync_copy(rows_v, out.at[pl.ds(off, CHUNK)])
      return carry

    lax.fori_loop(0, CPT, body, 0)

  return gk


def _make_scatter(nq):
  w = 128 * nq

  @functools.partial(
      pl.kernel,
      out_type=jax.ShapeDtypeStruct((NC, R2, w), _f32),
      mesh=_mesh(),
      scratch_types=[
          pltpu.VMEM((CHUNK,), jnp.int32),
          pltpu.VMEM((CHUNK, 128), _f32),
          pltpu.VMEM_SHARED((R2, 128), _f32),
      ],
  )
  def sk(vals, idx, zer, out, idx_v, buf, accum):
    cid = lax.axis_index("c")
    sid = lax.axis_index("s")
    base = _wid() * (CPT * CHUNK)
    rbase = sid * ROWS_PT

    for q in range(nq):
      pltpu.sync_copy(zer.at[pl.ds(rbase, ROWS_PT)],
                      accum.at[pl.ds(rbase, ROWS_PT)])
      plsc.subcore_barrier()

      def body(j, carry):
        off = base + j * CHUNK
        pltpu.sync_copy(idx.at[pl.ds(off, CHUNK)], idx_v)
        if nq == 1:
          pltpu.sync_copy(vals.at[pl.ds(off, CHUNK)], buf)
        else:
          pltpu.sync_copy(vals.at[pl.ds(off, CHUNK), pl.ds(q * 128, 128)],
                          buf)
        pltpu.sync_copy(buf, accum.at[idx_v], add=True)
        return carry

      lax.fori_loop(0, CPT, body, 0)
      plsc.subcore_barrier()
      if nq == 1:
        pltpu.sync_copy(accum.at[pl.ds(rbase, ROWS_PT)],
                        out.at[cid, pl.ds(rbase, ROWS_PT)])
      else:
        pltpu.sync_copy(accum.at[pl.ds(rbase, ROWS_PT)],
                        out.at[cid, pl.ds(rbase, ROWS_PT),
                               pl.ds(q * 128, 128)])
      plsc.subcore_barrier()

  return sk


_gather128 = _make_gather(128)
_gather512 = _make_gather(HC)
_scatter1 = _make_scatter(1)
_scatter4 = _make_scatter(4)


BN = 1000
BEDGE = 2048


def _expand_mat():
  row = lax.broadcasted_iota(jnp.int32, (H, HC), 0)
  col = lax.broadcasted_iota(jnp.int32, (H, HC), 1) // C
  return jnp.where(row == col, 1.0, 0.0).astype(_f32)


def _k1_body(x_ref, w1_ref, ad1_ref, w2_ref, ad2_ref, h1_ref, h2_ref, q_ref):
  xb = x_ref[...]
  st = _expand_mat().T
  h1 = jnp.dot(xb, w1_ref[...], preferred_element_type=_f32)
  h2 = jnp.dot(xb, w2_ref[...], preferred_element_type=_f32)
  h1_ref[...] = h1
  h2_ref[...] = h2
  a1d = jnp.dot(h1 * ad1_ref[...], st, preferred_element_type=_f32)
  a2d = jnp.dot(h2 * ad2_ref[...], st, preferred_element_type=_f32)
  q_ref[...] = jnp.concatenate(
      [a1d, a2d, jnp.zeros((BN, 128 - 2 * H), _f32)], axis=1)


def _k1(x, W1, ad1, W2, ad2):
  full = lambda shape: pl.BlockSpec(shape, lambda i: tuple(0 for _ in shape))
  return pl.pallas_call(
      _k1_body,
      grid=(N // BN,),
      in_specs=[
          pl.BlockSpec((BN, D), lambda i: (i, 0)),
          full((D, HC)), full((1, HC)),
          full((D, HC)), full((1, HC)),
      ],
      out_specs=[
          pl.BlockSpec((BN, HC), lambda i: (i, 0)),
          pl.BlockSpec((BN, HC), lambda i: (i, 0)),
          pl.BlockSpec((BN, 128), lambda i: (i, 0)),
      ],
      out_shape=[
          jax.ShapeDtypeStruct((N, HC), _f32),
          jax.ShapeDtypeStruct((N, HC), _f32),
          jax.ShapeDtypeStruct((N, 128), _f32),
      ],
  )(x, W1, ad1, W2, ad2)


def _k24_body(hg1_ref, ad1_ref, hg2_ref, ad2_ref, as1_ref, as2_ref,
              m1_ref, ex1_ref, m2_ref, ex2_ref):
  s = _expand_mat()
  st = s.T

  def one(hg, ad8, asf, m_ref, ex_ref):
    als = jnp.dot(hg * asf, st, preferred_element_type=_f32)
    z = als + ad8
    e = jnp.maximum(z, 0.2 * z)
    ex8 = jnp.exp(e)
    ex_ref[...] = jnp.concatenate(
        [ex8, jnp.zeros((BEDGE, 128 - H), _f32)], axis=1)
    m_ref[...] = hg * jnp.dot(ex8, s, preferred_element_type=_f32)

  one(hg1_ref[...], ad1_ref[...][:, 0:H], as1_ref[...], m1_ref, ex1_ref)
  one(hg2_ref[...], ad2_ref[...][:, H:2 * H], as2_ref[...], m2_ref, ex2_ref)


def _k24(hg1, Ad1, hg2, Ad2, as1, as2):
  full = lambda shape: pl.BlockSpec(shape, lambda i: tuple(0 for _ in shape))
  hspec = pl.BlockSpec((BEDGE, HC), lambda i: (i, 0))
  aspec = pl.BlockSpec((BEDGE, 128), lambda i: (i, 0))
  return pl.pallas_call(
      _k24_body,
      grid=(EEP // BEDGE,),
      in_specs=[hspec, aspec, hspec, aspec, full((1, HC)), full((1, HC))],
      out_specs=[hspec, aspec, hspec, aspec],
      out_shape=[
          jax.ShapeDtypeStruct((EEP, HC), _f32),
          jax.ShapeDtypeStruct((EEP, 128), _f32),
          jax.ShapeDtypeStruct((EEP, HC), _f32),
          jax.ShapeDtypeStruct((EEP, 128), _f32),
      ],
  )(hg1, Ad1, hg2, Ad2, as1, as2)


def _k5_body(pex1_ref, pm1_ref, pex2_ref, pm2_ref, oh_ref,
             bi1_ref, bi2_ref, wf1_ref, bf1_ref, wf2_ref, bf2_ref,
             w1_ref, b1_ref, w2_ref, b2_ref, w3_ref, b3_ref,
             gsum_ref, cnt_ref):
  s_exp = _expand_mat()

  def branch(pex, pm, bi, wf, bf):
    s = pex[0, :, 0:H] + pex[1, :, 0:H]
    agg = pm[0] + pm[1]
    den = jnp.dot(s, s_exp, preferred_element_type=_f32) + 1e-16
    n = agg / den + bi
    return jnp.dot(n, wf, preferred_element_type=_f32) + bf

  f1 = branch(pex1_ref[...], pm1_ref[...], bi1_ref[...], wf1_ref[...],
              bf1_ref[...])
  f2 = branch(pex2_ref[...], pm2_ref[...], bi2_ref[...], wf2_ref[...],
              bf2_ref[...])
  lk = lambda v: jnp.maximum(v, 0.1 * v)
  fused = lk(jnp.concatenate([f1, f2], axis=1))
  fused = lk(jnp.dot(fused, w1_ref[...], preferred_element_type=_f32)
             + b1_ref[...])
  fused = lk(jnp.dot(fused, w2_ref[...], preferred_element_type=_f32)
             + b2_ref[...])
  fused = lk(jnp.dot(fused, w3_ref[...], preferred_element_type=_f32)
             + b3_ref[...])
  oh = oh_ref[...]
  gs = lax.dot_general(oh, fused, (((0,), (0,)), ((), ())),
                       preferred_element_type=_f32)
  cn = lax.dot_general(oh, jnp.ones((BN, 32), _f32),
                       (((0,), (0,)), ((), ())),
                       preferred_element_type=_f32)
  i = pl.program_id(0)

  @pl.when(i == 0)
  def _():
    gsum_ref[...] = gs
    cnt_ref[...] = cn

  @pl.when(i > 0)
  def _():
    gsum_ref[...] += gs
    cnt_ref[...] += cn


def _k5(pex1, pm1, pex2, pm2, oh, bi1, bi2, wf1, bf1, wf2, bf2,
        w1, b1, w2, b2, w3, b3):
  full = lambda shape: pl.BlockSpec(shape, lambda i: tuple(0 for _ in shape))
  pex_spec = pl.BlockSpec((NC, BN, 128), lambda i: (0, i, 0))
  pm_spec = pl.BlockSpec((NC, BN, HC), lambda i: (0, i, 0))
  return pl.pallas_call(
      _k5_body,
      grid=(N // BN,),
      in_specs=[
          pex_spec, pm_spec, pex_spec, pm_spec,
          pl.BlockSpec((BN, G), lambda i: (i, 0)),
          full((1, HC)), full((1, HC)),
          full((HC, 128)), full((1, 128)), full((HC, 128)), full((1, 128)),
          full((256, 64)), full((1, 64)), full((64, 32)), full((1, 32)),
          full((32, 32)), full((1, 32)),
      ],
      out_specs=[full((G, 32)), full((G, 32))],
      out_shape=[jax.ShapeDtypeStruct((G, 32), _f32)] * 2,
  )(pex1, pm1, pex2, pm2, oh, bi1, bi2, wf1, bf1, wf2, bf2,
    w1, b1, w2, b2, w3, b3)


def _k6_body(gsum_ref, cnt_ref, we_ref, be_ref, wv_ref, bv_ref,
             oe_ref, ov_ref, l2_ref):
  g = gsum_ref[...] / jnp.maximum(cnt_ref[...], 1.0)
  oe_ref[...] = jnp.dot(g, we_ref[...], preferred_element_type=_f32) \
      + be_ref[...]
  ov_ref[...] = jnp.dot(g, wv_ref[...], preferred_element_type=_f32) \
      + bv_ref[...]
  l2_ref[...] = jnp.sum(g * g).reshape(1, 1) / (G * 32)


def _k6(gsum, cnt, We, be, Wv, bv):
  return pl.pallas_call(
      _k6_body,
      out_shape=[
          jax.ShapeDtypeStruct((G, 3), _f32),
          jax.ShapeDtypeStruct((G, 3), _f32),
          jax.ShapeDtypeStruct((1, 1), _f32),
      ],
  )(gsum, cnt, We, be, Wv, bv)


def kernel(x, edge_attr_ipv6, edge_attr_tsch, edge_index_ipv6,
           edge_index_tsch, batch, W_ipv6, a_src_ipv6, a_dst_ipv6, b_ipv6,
           W_tsch, a_src_tsch, a_dst_tsch, b_tsch, Wf_ipv6, bf_ipv6,
           Wf_tsch, bf_tsch, W1, b1, W2, b2, W3, b3, We, be, Wv, bv):
  ei1 = edge_index_ipv6.astype(jnp.int32)
  ei2 = edge_index_tsch.astype(jnp.int32)
  loop = jnp.arange(N, dtype=jnp.int32)
  padg = jnp.zeros((EEP - EE,), jnp.int32)
  pads = jnp.full((EEP - EE,), N, jnp.int32)

  src1 = jnp.concatenate([ei1[0], loop, padg])
  dst1g = jnp.concatenate([ei1[1], loop, padg])
  dst1s = jnp.concatenate([ei1[1], loop, pads])
  src2 = jnp.concatenate([ei2[0], loop, padg])
  dst2g = jnp.concatenate([ei2[1], loop, padg])
  dst2s = jnp.concatenate([ei2[1], loop, pads])

  r1 = lambda v: v.reshape(1, -1).astype(_f32)
  h1, h2, Q = _k1(x.astype(_f32), W_ipv6.astype(_f32), r1(a_dst_ipv6),
                  W_tsch.astype(_f32), r1(a_dst_tsch))

  hg1 = _gather512(h1, src1)
  hg2 = _gather512(h2, src2)
  Ad1 = _gather128(Q, dst1g)
  Ad2 = _gather128(Q, dst2g)

  m1, ex1, m2, ex2 = _k24(hg1, Ad1, hg2, Ad2,
                          r1(a_src_ipv6), r1(a_src_tsch))

  zer128 = jnp.zeros((R2, 128), _f32)
  pex1 = _scatter1(ex1, dst1s, zer128)
  pex2 = _scatter1(ex2, dst2s, zer128)
  pm1 = _scatter4(m1, dst1s, zer128)
  pm2 = _scatter4(m2, dst2s, zer128)

  oh = (batch.reshape(-1, 1) == jnp.arange(G, dtype=batch.dtype)
        .reshape(1, -1)).astype(_f32)
  gsum, cnt = _k5(pex1, pm1, pex2, pm2, oh,
                  r1(b_ipv6), r1(b_tsch),
                  Wf_ipv6.astype(_f32), r1(bf_ipv6),
                  Wf_tsch.astype(_f32), r1(bf_tsch),
                  W1.astype(_f32), r1(b1), W2.astype(_f32), r1(b2),
                  W3.astype(_f32), r1(b3))
  oe, ov, l2 = _k6(gsum, cnt, We.astype(_f32), r1(be),
                   Wv.astype(_f32), r1(bv))
  return (oe, ov, l2.reshape(()))
